# baseline (speedup 1.0000x reference)
import functools
import math

import jax
import jax.numpy as jnp
from jax import lax
from jax.experimental import pallas as pl
from jax.experimental.pallas import tpu as pltpu
from jax.experimental.pallas import tpu_sc as plsc

NC = 2
NS = 16
NW = NC * NS
LANES = 16
EB = 64
NBUF = 2
NH = 2


def _mesh():
    return plsc.VectorSubcoreMesh(
        core_axis_name="c", subcore_axis_name="s",
        num_cores=NC, num_subcores=NS)


def _zero_fill_2d(ref, nrows):
    ncol = ref.shape[1]

    def row(i, carry):
        for j in range(ncol // LANES):
            ref[i, pl.ds(j * LANES, LANES)] = jnp.zeros((LANES,), ref.dtype)
        return carry

    lax.fori_loop(0, nrows, row, 0)


def _deg_call(dst_p, np_, nb):
    npt = np_ // NS

    @functools.partial(
        pl.kernel,
        out_type=jax.ShapeDtypeStruct((NC, np_), jnp.float32),
        mesh=_mesh(),
        scratch_types=[
            pltpu.VMEM((nb, EB), jnp.int32),
            pltpu.VMEM((EB,), jnp.float32),
            pltpu.VMEM((npt,), jnp.float32),
            pltpu.VMEM_SHARED((np_,), jnp.float32),
        ],
    )
    def k(dst_hbm, deg_out, idx_v, ones_v, zbuf, deg_sh):
        c = lax.axis_index("c")
        s = lax.axis_index("s")
        wid = s * NC + c
        for j in range(EB // LANES):
            ones_v[pl.ds(j * LANES, LANES)] = jnp.ones((LANES,), jnp.float32)

        def zrow(i, carry):
            zbuf[pl.ds(i * LANES, LANES)] = jnp.zeros((LANES,), jnp.float32)
            return carry

        lax.fori_loop(0, npt // LANES, zrow, 0)
        pltpu.sync_copy(zbuf, deg_sh.at[pl.ds(s * npt, npt)])
        plsc.subcore_barrier()
        pltpu.sync_copy(dst_hbm.at[wid], idx_v)

        def body(i, carry):
            pltpu.sync_copy(ones_v, deg_sh.at[idx_v.at[i]], add=True)
            return carry

        lax.fori_loop(0, nb, body, 0)
        plsc.subcore_barrier()
        pltpu.sync_copy(deg_sh.at[pl.ds(s * npt, npt)],
                        deg_out.at[c, pl.ds(s * npt, npt)])

    return k(dst_p)


def _agg_call(g, src_p, dst_p, np_, nb):
    npt = np_ // NS
    hb = nb // NH

    @functools.partial(
        pl.kernel,
        out_type=jax.ShapeDtypeStruct((NC, np_, 128), jnp.float32),
        mesh=_mesh(),
        scratch_types=[
            pltpu.VMEM((hb, EB), jnp.int32),
            pltpu.VMEM((hb, EB), jnp.int32),
            [pltpu.VMEM((EB, 128), jnp.float32) for _ in range(NBUF)],
            [pltpu.SemaphoreType.DMA for _ in range(NBUF)],
            pltpu.VMEM_SHARED((np_, 128), jnp.float32),
        ],
    )
    def k(g_hbm, src_hbm, dst_hbm, part_out, idx_s, idx_d, rows, sems, acc):
        c = lax.axis_index("c")
        s = lax.axis_index("s")
        wid = s * NC + c
        _zero_fill_2d(rows[0], EB)
        for t in range(npt // EB):
            pltpu.sync_copy(rows[0], acc.at[pl.ds(s * npt + t * EB, EB)])
        rem = npt % EB
        if rem:
            pltpu.sync_copy(rows[0].at[pl.ds(0, rem)],
                            acc.at[pl.ds(s * npt + (npt // EB) * EB, rem)])
        plsc.subcore_barrier()

        for h in range(NH):
            pltpu.sync_copy(src_hbm.at[wid, pl.ds(h * hb, hb)], idx_s)
            pltpu.sync_copy(dst_hbm.at[wid, pl.ds(h * hb, hb)], idx_d)

            for b in range(NBUF):
                pltpu.async_copy(g_hbm.at[idx_s.at[b]], rows[b], sems[b])

            def body(gi, carry):
                for b in range(NBUF):
                    i = gi * NBUF + b
                    pltpu.make_async_copy(g_hbm.at[idx_s.at[i]],
                                          rows[b], sems[b]).wait()
                    pltpu.sync_copy(rows[b], acc.at[idx_d.at[i]], add=True)
                    pltpu.async_copy(g_hbm.at[idx_s.at[i + NBUF]],
                                     rows[b], sems[b])
                return carry

            lax.fori_loop(0, hb // NBUF - 1, body, 0)
            for b in range(NBUF):
                i = hb - NBUF + b
                pltpu.make_async_copy(g_hbm.at[idx_s.at[i]],
                                      rows[b], sems[b]).wait()
                pltpu.sync_copy(rows[b], acc.at[idx_d.at[i]], add=True)
        plsc.subcore_barrier()
        pltpu.sync_copy(acc.at[pl.ds(s * npt, npt)],
                        part_out.at[c, pl.ds(s * npt, npt)])

    return k(g, src_p, dst_p)


def _dinv_of(d_ref):
    d = d_ref[:, 0:1] + d_ref[:, 1:2] + 1.0
    return lax.rsqrt(d)


def _first_tc(x, w1, degt, blk):
    n = x.shape[0]

    def body(x_ref, w_ref, d_ref, o_ref):
        dinv = _dinv_of(d_ref)
        o_ref[...] = jnp.dot(x_ref[...], w_ref[...],
                             preferred_element_type=jnp.float32) * dinv

    return pl.pallas_call(
        body,
        grid=(n // blk,),
        in_specs=[
            pl.BlockSpec((blk, 128), lambda m: (m, 0)),
            pl.BlockSpec((128, 128), lambda m: (0, 0)),
            pl.BlockSpec((blk, NC), lambda m: (m, 0)),
        ],
        out_specs=pl.BlockSpec((blk, 128), lambda m: (m, 0)),
        out_shape=jax.ShapeDtypeStruct((n, 128), jnp.float32),
    )(x, w1, degt)


def _mid_tc(part, g1, degt, b1, w2, blk):
    n = g1.shape[0]

    def body(p_ref, g_ref, d_ref, b_ref, w_ref, o_ref):
        dinv = _dinv_of(d_ref)
        ssum = p_ref[0] + p_ref[1] + g_ref[...]
        h = jnp.maximum(ssum * dinv + b_ref[...], 0.0)
        o_ref[...] = jnp.dot(h, w_ref[...],
                             preferred_element_type=jnp.float32) * dinv

    return pl.pallas_call(
        body,
        grid=(n // blk,),
        in_specs=[
            pl.BlockSpec((NC, blk, 128), lambda m: (0, m, 0)),
            pl.BlockSpec((blk, 128), lambda m: (m, 0)),
            pl.BlockSpec((blk, NC), lambda m: (m, 0)),
            pl.BlockSpec((1, 128), lambda m: (0, 0)),
            pl.BlockSpec((128, 128), lambda m: (0, 0)),
        ],
        out_specs=pl.BlockSpec((blk, 128), lambda m: (m, 0)),
        out_shape=jax.ShapeDtypeStruct((n, 128), jnp.float32),
    )(part, g1, degt, b1, w2)


def _last_tc(part, g2, degt, b2, blk):
    n = g2.shape[0]

    def body(p_ref, g_ref, d_ref, b_ref, o_ref):
        dinv = _dinv_of(d_ref)
        ssum = p_ref[0] + p_ref[1] + g_ref[...]
        o_ref[...] = ssum * dinv + b_ref[...]

    return pl.pallas_call(
        body,
        grid=(n // blk,),
        in_specs=[
            pl.BlockSpec((NC, blk, 128), lambda m: (0, m, 0)),
            pl.BlockSpec((blk, 128), lambda m: (m, 0)),
            pl.BlockSpec((blk, NC), lambda m: (m, 0)),
            pl.BlockSpec((1, 128), lambda m: (0, 0)),
        ],
        out_specs=pl.BlockSpec((blk, 128), lambda m: (m, 0)),
        out_shape=jax.ShapeDtypeStruct((n, 128), jnp.float32),
    )(part, g2, degt, b2)


def kernel(x, edge_index, W1, b1, W2, b2):
    n = x.shape[0]
    e = edge_index.shape[1]
    blk = 2000 if n % 2000 == 0 else 8
    unit = NS * 2 * LANES
    np_deg = ((n + 1 + unit - 1) // unit) * unit
    np_agg = ((n + 1 + NS * 8 - 1) // (NS * 8)) * (NS * 8)
    nbu = NH * NBUF * (8 if NH > 1 else 1)
    nb = math.ceil(math.ceil(e / (NW * EB)) / nbu) * nbu
    epad = NW * EB * nb - e
    src = jnp.concatenate(
        [edge_index[0], jnp.zeros((epad,), edge_index.dtype)]).reshape(NW, nb, EB)
    dst = jnp.concatenate(
        [edge_index[1], jnp.full((epad,), n, edge_index.dtype)]).reshape(NW, nb, EB)

    deg = _deg_call(dst, np_deg, nb)
    degt = deg[:, :n].T
    b1r = b1.reshape(1, 128)
    b2r = b2.reshape(1, 128)

    g1 = _first_tc(x, W1, degt, blk)
    part1 = _agg_call(g1, src, dst, np_agg, nb)
    g2 = _mid_tc(part1, g1, degt, b1r, W2, blk)
    part2 = _agg_call(g2, src, dst, np_agg, nb)
    return _last_tc(part2, g2, degt, b2r, blk)


# speedup vs baseline: 2.0934x; 2.0934x over previous
import functools
import math

import jax
import jax.numpy as jnp
from jax import lax
from jax.experimental import pallas as pl
from jax.experimental.pallas import tpu as pltpu
from jax.experimental.pallas import tpu_sc as plsc

NC = 2
NS = 16
NW = NC * NS
LANES = 16
EB = 80
NBUF = 1
NH = 1


def _mesh():
    return plsc.VectorSubcoreMesh(
        core_axis_name="c", subcore_axis_name="s",
        num_cores=NC, num_subcores=NS)


def _zero_fill_2d(ref, nrows):
    ncol = ref.shape[1]

    def row(i, carry):
        for j in range(ncol // LANES):
            ref[i, pl.ds(j * LANES, LANES)] = jnp.zeros((LANES,), ref.dtype)
        return carry

    lax.fori_loop(0, nrows, row, 0)


def _deg_call(dst_p, np_, nb):
    npt = np_ // NS

    @functools.partial(
        pl.kernel,
        out_type=jax.ShapeDtypeStruct((NC, np_), jnp.float32),
        mesh=_mesh(),
        scratch_types=[
            pltpu.VMEM((nb, EB), jnp.int32),
            pltpu.VMEM((EB,), jnp.float32),
            pltpu.VMEM((npt,), jnp.float32),
            pltpu.VMEM_SHARED((np_,), jnp.float32),
        ],
    )
    def k(dst_hbm, deg_out, idx_v, ones_v, zbuf, deg_sh):
        c = lax.axis_index("c")
        s = lax.axis_index("s")
        wid = s * NC + c
        for j in range(EB // LANES):
            ones_v[pl.ds(j * LANES, LANES)] = jnp.ones((LANES,), jnp.float32)

        def zrow(i, carry):
            zbuf[pl.ds(i * LANES, LANES)] = jnp.zeros((LANES,), jnp.float32)
            return carry

        lax.fori_loop(0, npt // LANES, zrow, 0)
        pltpu.sync_copy(zbuf, deg_sh.at[pl.ds(s * npt, npt)])
        plsc.subcore_barrier()
        pltpu.sync_copy(dst_hbm.at[wid], idx_v)

        def body(i, carry):
            pltpu.sync_copy(ones_v, deg_sh.at[idx_v.at[i]], add=True)
            return carry

        lax.fori_loop(0, nb, body, 0)
        plsc.subcore_barrier()
        pltpu.sync_copy(deg_sh.at[pl.ds(s * npt, npt)],
                        deg_out.at[c, pl.ds(s * npt, npt)])

    return k(dst_p)


def _agg_call(g, src_p, dst_p, np_, nb):
    npt = np_ // NS
    hb = nb // NH

    @functools.partial(
        pl.kernel,
        out_type=jax.ShapeDtypeStruct((NC, np_, 128), jnp.float32),
        mesh=_mesh(),
        scratch_types=[
            pltpu.VMEM((hb, EB), jnp.int32),
            pltpu.VMEM((hb, EB), jnp.int32),
            [pltpu.VMEM((EB, 128), jnp.float32) for _ in range(NBUF)],
            [pltpu.SemaphoreType.DMA for _ in range(NBUF)],
            pltpu.VMEM_SHARED((np_, 128), jnp.float32),
        ],
    )
    def k(g_hbm, src_hbm, dst_hbm, part_out, idx_s, idx_d, rows, sems, acc):
        c = lax.axis_index("c")
        s = lax.axis_index("s")
        wid = s * NC + c
        _zero_fill_2d(rows[0], EB)
        for t in range(npt // EB):
            pltpu.sync_copy(rows[0], acc.at[pl.ds(s * npt + t * EB, EB)])
        rem = npt % EB
        if rem:
            pltpu.sync_copy(rows[0].at[pl.ds(0, rem)],
                            acc.at[pl.ds(s * npt + (npt // EB) * EB, rem)])
        plsc.subcore_barrier()

        for h in range(NH):
            pltpu.sync_copy(src_hbm.at[wid, pl.ds(h * hb, hb)], idx_s)
            pltpu.sync_copy(dst_hbm.at[wid, pl.ds(h * hb, hb)], idx_d)

            def body(i, carry):
                pltpu.async_copy(g_hbm.at[idx_s.at[i]], rows[0], sems[0]).wait()
                pltpu.sync_copy(rows[0], acc.at[idx_d.at[i]], add=True)
                return carry

            lax.fori_loop(0, hb, body, 0)
        plsc.subcore_barrier()
        pltpu.sync_copy(acc.at[pl.ds(s * npt, npt)],
                        part_out.at[c, pl.ds(s * npt, npt)])

    return k(g, src_p, dst_p)


def _dinv_of(d_ref):
    d = d_ref[:, 0:1] + d_ref[:, 1:2] + 1.0
    return lax.rsqrt(d)


def _first_tc(x, w1, degt, blk):
    n = x.shape[0]

    def body(x_ref, w_ref, d_ref, o_ref):
        dinv = _dinv_of(d_ref)
        o_ref[...] = jnp.dot(x_ref[...], w_ref[...],
                             preferred_element_type=jnp.float32) * dinv

    return pl.pallas_call(
        body,
        grid=(n // blk,),
        in_specs=[
            pl.BlockSpec((blk, 128), lambda m: (m, 0)),
            pl.BlockSpec((128, 128), lambda m: (0, 0)),
            pl.BlockSpec((blk, NC), lambda m: (m, 0)),
        ],
        out_specs=pl.BlockSpec((blk, 128), lambda m: (m, 0)),
        out_shape=jax.ShapeDtypeStruct((n, 128), jnp.float32),
    )(x, w1, degt)


def _mid_tc(part, g1, degt, b1, w2, blk):
    n = g1.shape[0]

    def body(p_ref, g_ref, d_ref, b_ref, w_ref, o_ref):
        dinv = _dinv_of(d_ref)
        ssum = p_ref[0] + p_ref[1] + g_ref[...]
        h = jnp.maximum(ssum * dinv + b_ref[...], 0.0)
        o_ref[...] = jnp.dot(h, w_ref[...],
                             preferred_element_type=jnp.float32) * dinv

    return pl.pallas_call(
        body,
        grid=(n // blk,),
        in_specs=[
            pl.BlockSpec((NC, blk, 128), lambda m: (0, m, 0)),
            pl.BlockSpec((blk, 128), lambda m: (m, 0)),
            pl.BlockSpec((blk, NC), lambda m: (m, 0)),
            pl.BlockSpec((1, 128), lambda m: (0, 0)),
            pl.BlockSpec((128, 128), lambda m: (0, 0)),
        ],
        out_specs=pl.BlockSpec((blk, 128), lambda m: (m, 0)),
        out_shape=jax.ShapeDtypeStruct((n, 128), jnp.float32),
    )(part, g1, degt, b1, w2)


def _last_tc(part, g2, degt, b2, blk):
    n = g2.shape[0]

    def body(p_ref, g_ref, d_ref, b_ref, o_ref):
        dinv = _dinv_of(d_ref)
        ssum = p_ref[0] + p_ref[1] + g_ref[...]
        o_ref[...] = ssum * dinv + b_ref[...]

    return pl.pallas_call(
        body,
        grid=(n // blk,),
        in_specs=[
            pl.BlockSpec((NC, blk, 128), lambda m: (0, m, 0)),
            pl.BlockSpec((blk, 128), lambda m: (m, 0)),
            pl.BlockSpec((blk, NC), lambda m: (m, 0)),
            pl.BlockSpec((1, 128), lambda m: (0, 0)),
        ],
        out_specs=pl.BlockSpec((blk, 128), lambda m: (m, 0)),
        out_shape=jax.ShapeDtypeStruct((n, 128), jnp.float32),
    )(part, g2, degt, b2)


def kernel(x, edge_index, W1, b1, W2, b2):
    n = x.shape[0]
    e = edge_index.shape[1]
    blk = 2000 if n % 2000 == 0 else 8
    unit = NS * 2 * LANES
    np_deg = ((n + 1 + unit - 1) // unit) * unit
    np_agg = ((n + 1 + NS * 8 - 1) // (NS * 8)) * (NS * 8)
    nbu = NH * NBUF * (8 if NH > 1 else 1)
    nb = math.ceil(math.ceil(e / (NW * EB)) / nbu) * nbu
    epad = NW * EB * nb - e
    src = jnp.concatenate(
        [edge_index[0], jnp.zeros((epad,), edge_index.dtype)]).reshape(NW, nb, EB)
    dst = jnp.concatenate(
        [edge_index[1], jnp.full((epad,), n, edge_index.dtype)]).reshape(NW, nb, EB)

    deg = _deg_call(dst, np_deg, nb)
    degt = deg[:, :n].T
    b1r = b1.reshape(1, 128)
    b2r = b2.reshape(1, 128)

    g1 = _first_tc(x, W1, degt, blk)
    part1 = _agg_call(g1, src, dst, np_agg, nb)
    g2 = _mid_tc(part1, g1, degt, b1r, W2, blk)
    part2 = _agg_call(g2, src, dst, np_agg, nb)
    return _last_tc(part2, g2, degt, b2r, blk)
